# baseline (speedup 1.0000x reference)
import functools

import jax
import jax.numpy as jnp
import numpy as np
from jax import lax
from jax.experimental import pallas as pl
from jax.experimental.pallas import tpu as pltpu
from jax.experimental.pallas import tpu_sc as plsc

N = 16384
NB = 1000
SMOOTH = 0.1
COLD = np.float32(SMOOTH / NB)
HOT = np.float32(np.float32(1.0 - SMOOTH) + COLD)

LANES = 16
CA = 16
CB = NB - CA
TC_BLK = 1024


def _build_sc_call(num_cores: int, num_subcores: int):
    num_workers = num_cores * num_subcores
    cols_per_w = N // num_workers
    mesh = plsc.VectorSubcoreMesh(
        core_axis_name="c", subcore_axis_name="s",
        num_cores=num_cores, num_subcores=num_subcores)

    @functools.partial(
        pl.kernel,
        out_type=jax.ShapeDtypeStruct((CA, N), jnp.float32),
        mesh=mesh,
        scratch_types=[
            pltpu.VMEM((cols_per_w,), jnp.int32),
            pltpu.VMEM((CA, cols_per_w), jnp.float32),
        ],
        compiler_params=pltpu.CompilerParams(needs_layout_passes=False),
    )
    def sc_kernel(x_hbm, fill_hbm, out_hbm, lab_v, buf_v):
        wid = lax.axis_index("s") * num_cores + lax.axis_index("c")
        base_col = wid * cols_per_w
        pltpu.sync_copy(x_hbm.at[pl.ds(base_col, cols_per_w)], lab_v)
        pltpu.sync_copy(fill_hbm, buf_v)

        hot = jnp.full((LANES,), HOT, jnp.float32)
        lane = lax.iota(jnp.int32, LANES)
        for j in range(cols_per_w // LANES):
            labs = lab_v[pl.ds(j * LANES, LANES)]
            mask = labs >= CB
            rows = jnp.maximum(labs, CB) - CB
            plsc.store_scatter(buf_v, [rows, lane + j * LANES], hot, mask=mask)
        pltpu.sync_copy(buf_v, out_hbm.at[:, pl.ds(base_col, cols_per_w)])

    return sc_kernel


def _tc_body(x_ref, out_ref):
    labs = x_ref[0, 0, :]
    rows = lax.broadcasted_iota(jnp.int32, (CB, TC_BLK), 0)
    out_ref[:, :] = jnp.where(rows == labs[None, :], HOT, COLD)


def kernel(x):
    xi = x.astype(jnp.int32)
    info = plsc.get_sparse_core_info()
    sc_call = _build_sc_call(info.num_cores, info.num_subcores)
    fill = np.full((CA, N // (info.num_cores * info.num_subcores)), COLD,
                   np.float32)
    sc_part = sc_call(xi, fill)

    tc_full = pl.pallas_call(
        _tc_body,
        out_shape=jax.ShapeDtypeStruct((NB, N), jnp.float32),
        grid=(N // TC_BLK,),
        in_specs=[pl.BlockSpec((1, 1, TC_BLK), lambda j: (j, 0, 0))],
        out_specs=pl.BlockSpec((CB, TC_BLK), lambda j: (0, j)),
    )(xi.reshape(N // TC_BLK, 1, TC_BLK))

    return lax.dynamic_update_slice(tc_full, sc_part, (CB, 0)).T


# speedup vs baseline: 1.0107x; 1.0107x over previous
import functools

import jax
import jax.numpy as jnp
import numpy as np
from jax import lax
from jax.experimental import pallas as pl
from jax.experimental.pallas import tpu as pltpu
from jax.experimental.pallas import tpu_sc as plsc

N = 16384
NB = 1000
SMOOTH = 0.1
COLD = np.float32(SMOOTH / NB)
HOT = np.float32(np.float32(1.0 - SMOOTH) + COLD)

LANES = 16
CA = 8
CB = NB - CA
TC_BLK = 1024


def _build_sc_call(num_cores: int, num_subcores: int):
    num_workers = num_cores * num_subcores
    cols_per_w = N // num_workers
    mesh = plsc.VectorSubcoreMesh(
        core_axis_name="c", subcore_axis_name="s",
        num_cores=num_cores, num_subcores=num_subcores)

    @functools.partial(
        pl.kernel,
        out_type=jax.ShapeDtypeStruct((CA, N), jnp.float32),
        mesh=mesh,
        scratch_types=[
            pltpu.VMEM((cols_per_w,), jnp.int32),
            pltpu.VMEM((CA, cols_per_w), jnp.float32),
        ],
        compiler_params=pltpu.CompilerParams(needs_layout_passes=False),
    )
    def sc_kernel(x_hbm, fill_hbm, out_hbm, lab_v, buf_v):
        wid = lax.axis_index("s") * num_cores + lax.axis_index("c")
        base_col = wid * cols_per_w
        pltpu.sync_copy(x_hbm.at[pl.ds(base_col, cols_per_w)], lab_v)
        pltpu.sync_copy(fill_hbm, buf_v)

        hot = jnp.full((LANES,), HOT, jnp.float32)
        lane = lax.iota(jnp.int32, LANES)
        for j in range(cols_per_w // LANES):
            labs = lab_v[pl.ds(j * LANES, LANES)]
            mask = labs >= CB
            rows = jnp.maximum(labs, CB) - CB
            plsc.store_scatter(buf_v, [rows, lane + j * LANES], hot, mask=mask)
        pltpu.sync_copy(buf_v, out_hbm.at[:, pl.ds(base_col, cols_per_w)])

    return sc_kernel


def _tc_body(x_ref, out_ref):
    labs = x_ref[0, 0, :]
    rows = lax.broadcasted_iota(jnp.int32, (CB, TC_BLK), 0)
    out_ref[:, :] = jnp.where(rows == labs[None, :], HOT, COLD)


def kernel(x):
    xi = x.astype(jnp.int32)
    info = plsc.get_sparse_core_info()
    sc_call = _build_sc_call(info.num_cores, info.num_subcores)
    fill = np.full((CA, N // (info.num_cores * info.num_subcores)), COLD,
                   np.float32)
    sc_part = sc_call(xi, fill)

    tc_full = pl.pallas_call(
        _tc_body,
        out_shape=jax.ShapeDtypeStruct((NB, N), jnp.float32),
        grid=(N // TC_BLK,),
        in_specs=[pl.BlockSpec((1, 1, TC_BLK), lambda j: (j, 0, 0))],
        out_specs=pl.BlockSpec((CB, TC_BLK), lambda j: (0, j)),
    )(xi.reshape(N // TC_BLK, 1, TC_BLK))

    return lax.dynamic_update_slice(tc_full, sc_part, (CB, 0)).T
